# baseline (speedup 1.0000x reference)
import functools

import jax
import jax.numpy as jnp
from jax import lax
from jax.experimental import pallas as pl
from jax.experimental.pallas import tpu as pltpu
from jax.experimental.pallas import tpu_sc as plsc

N = 50000
E = 800000
F = 100
FP = 128
H1 = 256
L = 19
LP = 128

NT = 32
EPT = 25600
EP = NT * EPT
NP = 51200
R = 512
NCHUNK = 16
CW = 8

BA = 128
NB = EPT // BA
NPT = NP // 16

_INTERPRET = False


def _pcall(body, grid, in_specs, out_specs, out_shape):
    return pl.pallas_call(
        body, grid=grid, in_specs=in_specs, out_specs=out_specs,
        out_shape=out_shape, interpret=_INTERPRET)


def _set_ones_col(h):
    lane = lax.broadcasted_iota(jnp.int32, h.shape, 1)
    return jnp.where(lane == 100, 1.0, h)


def _epilogue(u0_ref, u1_ref, sd_ref, h_ref, b_ref):
    h = h_ref[...]
    U = u0_ref[...] + u1_ref[...]
    s = sd_ref[:, 0]
    d = sd_ref[:, 1]
    e = s + d
    p_self = jnp.exp(jnp.where(e > 0, e, 0.2 * e))
    q = U + p_self[:, None] * h
    z = q[:, 100:101] + 1e-16
    return jnp.maximum(q / z + b_ref[...], 0.0)


def _tc1_body(x_ref, w_ref, a_ref, h_ref, sd_ref):
    xb = jnp.clip(x_ref[...], -0.4, 0.4)
    h = jnp.dot(xb, w_ref[...], preferred_element_type=jnp.float32)
    sd_ref[...] = jnp.dot(h, a_ref[...], preferred_element_type=jnp.float32)
    h_ref[...] = _set_ones_col(h)


def _tc1(x_pad, Wp, Ap):
    grid = (NP // R,)
    in_specs = [
        pl.BlockSpec((R, FP), lambda i: (i, 0)),
        pl.BlockSpec((FP, FP), lambda i: (0, 0)),
        pl.BlockSpec((FP, 8), lambda i: (0, 0)),
    ]
    out_specs = [pl.BlockSpec((R, FP), lambda i: (i, 0)),
                 pl.BlockSpec((R, 8), lambda i: (i, 0))]
    out_shape = [jax.ShapeDtypeStruct((NP, FP), jnp.float32),
                 jax.ShapeDtypeStruct((NP, 8), jnp.float32)]
    return _pcall(_tc1_body, grid, in_specs, out_specs, out_shape)(
        x_pad, Wp, Ap)


def _tc_epi_body(u0_ref, u1_ref, sd_ref, h_ref, b_ref, w_ref, a_ref,
                 oh_ref, osd_ref):
    g = _epilogue(u0_ref, u1_ref, sd_ref, h_ref, b_ref)
    hn = jnp.dot(g, w_ref[...], preferred_element_type=jnp.float32)
    osd_ref[...] = jnp.dot(hn, a_ref[...], preferred_element_type=jnp.float32)
    oh_ref[...] = _set_ones_col(hn)


def _tc_epi(U2, sd, h_full, bp, Wp, Ap):
    grid = (NP // R,)
    in_specs = [
        pl.BlockSpec((R, FP), lambda i: (i, 0)),
        pl.BlockSpec((R, FP), lambda i: (i, 0)),
        pl.BlockSpec((R, 8), lambda i: (i, 0)),
        pl.BlockSpec((R, FP), lambda i: (i, 0)),
        pl.BlockSpec((1, FP), lambda i: (0, 0)),
        pl.BlockSpec((FP, FP), lambda i: (0, 0)),
        pl.BlockSpec((FP, 8), lambda i: (0, 0)),
    ]
    out_specs = [pl.BlockSpec((R, FP), lambda i: (i, 0)),
                 pl.BlockSpec((R, 8), lambda i: (i, 0))]
    out_shape = [jax.ShapeDtypeStruct((NP, FP), jnp.float32),
                 jax.ShapeDtypeStruct((NP, 8), jnp.float32)]
    return _pcall(_tc_epi_body, grid, in_specs, out_specs, out_shape)(
        U2[0], U2[1], sd, h_full, bp, Wp, Ap)


def _tc_fin_body(u0_ref, u1_ref, sd_ref, h_ref, b_ref, wf1_ref, bf1_ref,
                 wf2_ref, bf2_ref, out_ref):
    g = _epilogue(u0_ref, u1_ref, sd_ref, h_ref, b_ref)
    t = jnp.dot(g, wf1_ref[...], preferred_element_type=jnp.float32)
    t = jnp.maximum(t + bf1_ref[...], 0.0)
    lg = jnp.dot(t, wf2_ref[...], preferred_element_type=jnp.float32)
    lg = lg + bf2_ref[...]
    m = jnp.max(lg, axis=1, keepdims=True)
    ls = lg - m
    lse = jnp.log(jnp.sum(jnp.exp(ls), axis=1, keepdims=True))
    out_ref[...] = ls - lse


def _tc_fin(U2, sd, h_full, bp, Wf1p, bf1p, Wf2p, bf2p):
    grid = (NP // R,)
    in_specs = [
        pl.BlockSpec((R, FP), lambda i: (i, 0)),
        pl.BlockSpec((R, FP), lambda i: (i, 0)),
        pl.BlockSpec((R, 8), lambda i: (i, 0)),
        pl.BlockSpec((R, FP), lambda i: (i, 0)),
        pl.BlockSpec((1, FP), lambda i: (0, 0)),
        pl.BlockSpec((FP, H1), lambda i: (0, 0)),
        pl.BlockSpec((1, H1), lambda i: (0, 0)),
        pl.BlockSpec((H1, LP), lambda i: (0, 0)),
        pl.BlockSpec((1, LP), lambda i: (0, 0)),
    ]
    out_specs = pl.BlockSpec((R, LP), lambda i: (i, 0))
    out_shape = jax.ShapeDtypeStruct((NP, LP), jnp.float32)
    return _pcall(_tc_fin_body, grid, in_specs, out_specs, out_shape)(
        U2[0], U2[1], sd, h_full, bp, Wf1p, bf1p, Wf2p, bf2p)


def _sc_mesh():
    return plsc.VectorSubcoreMesh(
        core_axis_name="c", subcore_axis_name="s", num_cores=2,
        num_subcores=16)


_SC_PARAMS = pltpu.CompilerParams(
    needs_layout_passes=False, use_tc_tiling_on_sc=False)


def _edge_attention(s, d, src2, dst2):
    @functools.partial(
        pl.kernel,
        out_type=jax.ShapeDtypeStruct((EP // BA, BA), jnp.float32),
        mesh=_sc_mesh(),
        compiler_params=_SC_PARAMS,
        scratch_types=[
            pltpu.VMEM((NP,), jnp.float32),
            pltpu.VMEM((NP,), jnp.float32),
            pltpu.VMEM((BA,), jnp.int32),
            pltpu.VMEM((BA,), jnp.int32),
            pltpu.VMEM((BA,), jnp.float32),
        ],
    )
    def att(s_hbm, d_hbm, src_hbm, dst_hbm, p_hbm,
            s_v, d_v, src_v, dst_v, p_v):
        cid = lax.axis_index("c")
        sid = lax.axis_index("s")
        wid = sid * 2 + cid
        row0 = wid * NB
        pltpu.sync_copy(s_hbm, s_v)
        pltpu.sync_copy(d_hbm, d_v)

        def batch(bi, _):
            pltpu.sync_copy(src_hbm.at[row0 + bi], src_v)
            pltpu.sync_copy(dst_hbm.at[row0 + bi], dst_v)
            for g in range(BA // 16):
                si = src_v[pl.ds(g * 16, 16)]
                di = dst_v[pl.ds(g * 16, 16)]
                sv = plsc.load_gather(s_v, [si])
                dv = plsc.load_gather(d_v, [di])
                e = sv + dv
                e = jnp.where(e > 0.0, e, 0.2 * e)
                p_v[pl.ds(g * 16, 16)] = jnp.exp(e)
            pltpu.sync_copy(p_v, p_hbm.at[row0 + bi])
            return ()
        lax.fori_loop(0, NB, batch, ())

    return att(s, d, src2, dst2)


def _edge_aggregate(h_int, p2, src16_2, dst2):
    @functools.partial(
        pl.kernel,
        out_type=[jax.ShapeDtypeStruct((NP * NCHUNK, CW), jnp.float32),
                  jax.ShapeDtypeStruct((NP * NCHUNK, CW), jnp.float32)],
        mesh=_sc_mesh(),
        compiler_params=_SC_PARAMS,
        scratch_types=[
            pltpu.VMEM((NB, BA), jnp.int32),
            pltpu.VMEM((NB, BA), jnp.int32),
            pltpu.VMEM((NB, BA), jnp.float32),
            pltpu.VMEM((BA, CW), jnp.float32),
            pltpu.VMEM((BA, CW), jnp.float32),
            pltpu.VMEM((BA, CW), jnp.float32),
            pltpu.VMEM((BA, CW), jnp.float32),
            pltpu.VMEM((BA,), jnp.int32),
            pltpu.VMEM_SHARED((NP, CW), jnp.float32),
            pltpu.SemaphoreType.DMA,
            pltpu.SemaphoreType.DMA,
            pltpu.SemaphoreType.DMA,
            pltpu.SemaphoreType.DMA,
        ],
    )
    def agg(h_hbm, src_hbm, dst_hbm, p_hbm, u0_hbm, u1_hbm,
            src_v, dst_v, p_v, rows0_v, rows1_v, zz_v, stage_v, idx_v,
            u_sp, sem0, sem1, semS0, semS1):
        cid = lax.axis_index("c")
        sid = lax.axis_index("s")
        wid = sid * 2 + cid
        row0 = wid * NB
        lane = lax.iota(jnp.int32, 16)
        hi8 = (lane >= 8).astype(jnp.int32)
        cidx = lane % 8
        pltpu.sync_copy(src_hbm.at[pl.ds(row0, NB)], src_v)
        pltpu.sync_copy(dst_hbm.at[pl.ds(row0, NB)], dst_v)
        pltpu.sync_copy(p_hbm.at[pl.ds(row0, NB)], p_v)
        zero16 = jnp.zeros((16,), jnp.float32)
        for k in range(BA // 2):
            plsc.store_scatter(zz_v, [2 * k + hi8, cidx], zero16)

        def scale(bi, rows_v):
            bvec = jnp.full((16,), bi, jnp.int32)
            for k in range(BA // 2):
                ridx = 2 * k + hi8
                pv = plsc.load_gather(p_v, [bvec, ridx])
                v = plsc.load_gather(rows_v, [ridx, cidx])
                plsc.store_scatter(rows_v, [ridx, cidx], v * pv)

        def chunk(c, _):
            def zinit(i, _):
                pltpu.sync_copy(zz_v, u_sp.at[pl.ds(sid * NPT + i * BA, BA)])
                return ()
            lax.fori_loop(0, NPT // BA, zinit, ())
            plsc.subcore_barrier()

            pltpu.async_copy(h_hbm.at[src_v.at[0]], rows0_v, sem0)
            pltpu.async_copy(h_hbm.at[src_v.at[1]], rows1_v, sem1)

            def pair(k, _):
                b0 = 2 * k
                pltpu.make_async_copy(h_hbm.at[src_v.at[b0]], rows0_v,
                                      sem0).wait()
                scale(b0, rows0_v)
                pltpu.async_copy(rows0_v, u_sp.at[dst_v.at[b0]], semS0,
                                 add=True)
                pltpu.make_async_copy(h_hbm.at[src_v.at[b0 + 1]], rows1_v,
                                      sem1).wait()
                scale(b0 + 1, rows1_v)
                pltpu.async_copy(rows1_v, u_sp.at[dst_v.at[b0 + 1]], semS1,
                                 add=True)
                nxt0 = jnp.where(b0 + 2 >= NB, 0, b0 + 2)
                nxt1 = jnp.where(b0 + 3 >= NB, 1, b0 + 3)
                pltpu.make_async_copy(rows0_v, u_sp.at[dst_v.at[b0]],
                                      semS0).wait()
                pltpu.async_copy(h_hbm.at[src_v.at[nxt0]], rows0_v, sem0)
                pltpu.make_async_copy(rows1_v, u_sp.at[dst_v.at[b0 + 1]],
                                      semS1).wait()
                pltpu.async_copy(h_hbm.at[src_v.at[nxt1]], rows1_v, sem1)
                return ()
            lax.fori_loop(0, NB // 2, pair, ())
            pltpu.make_async_copy(h_hbm.at[src_v.at[0]], rows0_v,
                                  sem0).wait()
            pltpu.make_async_copy(h_hbm.at[src_v.at[1]], rows1_v,
                                  sem1).wait()
            plsc.subcore_barrier()

            def dump(i, _):
                node0 = sid * NPT + i * BA
                pltpu.sync_copy(u_sp.at[pl.ds(node0, BA)], stage_v)
                for g in range(BA // 16):
                    idx_v[pl.ds(g * 16, 16)] = (
                        (node0 + g * 16 + lane) * NCHUNK + c)
                @pl.when(cid == 0)
                def _():
                    pltpu.sync_copy(stage_v, u0_hbm.at[idx_v])
                @pl.when(cid == 1)
                def _():
                    pltpu.sync_copy(stage_v, u1_hbm.at[idx_v])
                return ()
            lax.fori_loop(0, NPT // BA, dump, ())
            plsc.subcore_barrier()

            def bump(i, _):
                r = i // (BA // 16)
                g = i % (BA // 16)
                src_v[r, pl.ds(g * 16, 16)] = src_v[r, pl.ds(g * 16, 16)] + 1
                return ()
            lax.fori_loop(0, NB * (BA // 16), bump, ())
            return ()
        lax.fori_loop(0, NCHUNK, chunk, ())

    return agg(h_int, src16_2, dst2, p2)


def kernel(x, edge_index, W1, asrc1, adst1, b1, W2, asrc2, adst2, b2,
           Wf1, bf1, Wf2, bf2):
    f32 = jnp.float32
    x_pad = jnp.zeros((NP, FP), f32).at[:N, :F].set(x)
    W1p = jnp.zeros((FP, FP), f32).at[:F, :F].set(W1)
    W2p = jnp.zeros((FP, FP), f32).at[:F, :F].set(W2)
    A1 = jnp.zeros((FP, 8), f32).at[:F, 0].set(asrc1).at[:F, 1].set(adst1)
    A2 = jnp.zeros((FP, 8), f32).at[:F, 0].set(asrc2).at[:F, 1].set(adst2)
    b1p = jnp.zeros((1, FP), f32).at[0, :F].set(b1)
    b2p = jnp.zeros((1, FP), f32).at[0, :F].set(b2)
    Wf1p = jnp.zeros((FP, H1), f32).at[:F, :].set(Wf1)
    bf1p = bf1.reshape(1, H1)
    Wf2p = jnp.zeros((H1, LP), f32).at[:, :L].set(Wf2)
    bf2p = jnp.full((1, LP), -1e30, f32).at[0, :L].set(bf2)

    npad = EP - E
    pad_idx = (N + (jnp.arange(npad, dtype=jnp.int32) % (NP - N)))
    srcp = jnp.concatenate([edge_index[0], pad_idx])
    src2 = srcp.reshape(EP // BA, BA)
    src16_2 = (srcp * NCHUNK).reshape(EP // BA, BA)
    dst2 = jnp.concatenate([edge_index[1], pad_idx]).reshape(EP // BA, BA)

    def ureshape(us):
        return (us[0].reshape(NP, FP), us[1].reshape(NP, FP))

    h1_full, sd1 = _tc1(x_pad, W1p, A1)
    p1 = _edge_attention(sd1[:, 0], sd1[:, 1], src2, dst2)
    U1 = _edge_aggregate(h1_full.reshape(NP * NCHUNK, CW), p1, src16_2, dst2)

    h2_full, sd2 = _tc_epi(ureshape(U1), sd1, h1_full, b1p, W2p, A2)
    p2 = _edge_attention(sd2[:, 0], sd2[:, 1], src2, dst2)
    U2 = _edge_aggregate(h2_full.reshape(NP * NCHUNK, CW), p2, src16_2, dst2)

    out_full = _tc_fin(ureshape(U2), sd2, h2_full, b2p, Wf1p, bf1p, Wf2p,
                       bf2p)
    return out_full[:N, :L]


# speedup vs baseline: 1.1367x; 1.1367x over previous
import functools

import jax
import jax.numpy as jnp
from jax import lax
from jax.experimental import pallas as pl
from jax.experimental.pallas import tpu as pltpu
from jax.experimental.pallas import tpu_sc as plsc

N = 50000
E = 800000
F = 100
FP = 128
H1 = 256
L = 19
LP = 128

NT = 32
EPT = 25600
EP = NT * EPT
NP = 51200
R = 512
NCHUNK = 16
CW = 8

BA = 128
NB = EPT // BA
NPT = NP // 16

_INTERPRET = False


def _pcall(body, grid, in_specs, out_specs, out_shape):
    return pl.pallas_call(
        body, grid=grid, in_specs=in_specs, out_specs=out_specs,
        out_shape=out_shape, interpret=_INTERPRET)


def _set_ones_col(h):
    lane = lax.broadcasted_iota(jnp.int32, h.shape, 1)
    return jnp.where(lane == 100, 1.0, h)


def _epilogue(u0_ref, u1_ref, sd_ref, h_ref, b_ref):
    h = h_ref[...]
    U = u0_ref[...] + u1_ref[...]
    s = sd_ref[:, 0]
    d = sd_ref[:, 1]
    e = s + d
    p_self = jnp.exp(jnp.where(e > 0, e, 0.2 * e))
    q = U + p_self[:, None] * h
    z = q[:, 100:101] + 1e-16
    return jnp.maximum(q / z + b_ref[...], 0.0)


def _tc1_body(x_ref, w_ref, a_ref, h_ref, sd_ref):
    xb = jnp.clip(x_ref[...], -0.4, 0.4)
    h = jnp.dot(xb, w_ref[...], preferred_element_type=jnp.float32)
    sd_ref[...] = jnp.dot(h, a_ref[...], preferred_element_type=jnp.float32)
    h_ref[...] = _set_ones_col(h)


def _tc1(x_pad, Wp, Ap):
    grid = (NP // R,)
    in_specs = [
        pl.BlockSpec((R, FP), lambda i: (i, 0)),
        pl.BlockSpec((FP, FP), lambda i: (0, 0)),
        pl.BlockSpec((FP, 8), lambda i: (0, 0)),
    ]
    out_specs = [pl.BlockSpec((R, FP), lambda i: (i, 0)),
                 pl.BlockSpec((R, 8), lambda i: (i, 0))]
    out_shape = [jax.ShapeDtypeStruct((NP, FP), jnp.float32),
                 jax.ShapeDtypeStruct((NP, 8), jnp.float32)]
    return _pcall(_tc1_body, grid, in_specs, out_specs, out_shape)(
        x_pad, Wp, Ap)


def _tc_epi_body(u0_ref, u1_ref, sd_ref, h_ref, b_ref, w_ref, a_ref,
                 oh_ref, osd_ref):
    g = _epilogue(u0_ref, u1_ref, sd_ref, h_ref, b_ref)
    hn = jnp.dot(g, w_ref[...], preferred_element_type=jnp.float32)
    osd_ref[...] = jnp.dot(hn, a_ref[...], preferred_element_type=jnp.float32)
    oh_ref[...] = _set_ones_col(hn)


def _tc_epi(U2, sd, h_full, bp, Wp, Ap):
    grid = (NP // R,)
    in_specs = [
        pl.BlockSpec((R, FP), lambda i: (i, 0)),
        pl.BlockSpec((R, FP), lambda i: (i, 0)),
        pl.BlockSpec((R, 8), lambda i: (i, 0)),
        pl.BlockSpec((R, FP), lambda i: (i, 0)),
        pl.BlockSpec((1, FP), lambda i: (0, 0)),
        pl.BlockSpec((FP, FP), lambda i: (0, 0)),
        pl.BlockSpec((FP, 8), lambda i: (0, 0)),
    ]
    out_specs = [pl.BlockSpec((R, FP), lambda i: (i, 0)),
                 pl.BlockSpec((R, 8), lambda i: (i, 0))]
    out_shape = [jax.ShapeDtypeStruct((NP, FP), jnp.float32),
                 jax.ShapeDtypeStruct((NP, 8), jnp.float32)]
    return _pcall(_tc_epi_body, grid, in_specs, out_specs, out_shape)(
        U2[0], U2[1], sd, h_full, bp, Wp, Ap)


def _tc_fin_body(u0_ref, u1_ref, sd_ref, h_ref, b_ref, wf1_ref, bf1_ref,
                 wf2_ref, bf2_ref, out_ref):
    g = _epilogue(u0_ref, u1_ref, sd_ref, h_ref, b_ref)
    t = jnp.dot(g, wf1_ref[...], preferred_element_type=jnp.float32)
    t = jnp.maximum(t + bf1_ref[...], 0.0)
    lg = jnp.dot(t, wf2_ref[...], preferred_element_type=jnp.float32)
    lg = lg + bf2_ref[...]
    m = jnp.max(lg, axis=1, keepdims=True)
    ls = lg - m
    lse = jnp.log(jnp.sum(jnp.exp(ls), axis=1, keepdims=True))
    out_ref[...] = ls - lse


def _tc_fin(U2, sd, h_full, bp, Wf1p, bf1p, Wf2p, bf2p):
    grid = (NP // R,)
    in_specs = [
        pl.BlockSpec((R, FP), lambda i: (i, 0)),
        pl.BlockSpec((R, FP), lambda i: (i, 0)),
        pl.BlockSpec((R, 8), lambda i: (i, 0)),
        pl.BlockSpec((R, FP), lambda i: (i, 0)),
        pl.BlockSpec((1, FP), lambda i: (0, 0)),
        pl.BlockSpec((FP, H1), lambda i: (0, 0)),
        pl.BlockSpec((1, H1), lambda i: (0, 0)),
        pl.BlockSpec((H1, LP), lambda i: (0, 0)),
        pl.BlockSpec((1, LP), lambda i: (0, 0)),
    ]
    out_specs = pl.BlockSpec((R, LP), lambda i: (i, 0))
    out_shape = jax.ShapeDtypeStruct((NP, LP), jnp.float32)
    return _pcall(_tc_fin_body, grid, in_specs, out_specs, out_shape)(
        U2[0], U2[1], sd, h_full, bp, Wf1p, bf1p, Wf2p, bf2p)


def _sc_mesh():
    return plsc.VectorSubcoreMesh(
        core_axis_name="c", subcore_axis_name="s", num_cores=2,
        num_subcores=16)


_SC_PARAMS = pltpu.CompilerParams(
    needs_layout_passes=False, use_tc_tiling_on_sc=False)


def _edge_attention(s, d, src2, dst2):
    @functools.partial(
        pl.kernel,
        out_type=jax.ShapeDtypeStruct((EP // BA, BA), jnp.float32),
        mesh=_sc_mesh(),
        compiler_params=_SC_PARAMS,
        scratch_types=[
            pltpu.VMEM((NP,), jnp.float32),
            pltpu.VMEM((NP,), jnp.float32),
            pltpu.VMEM((BA,), jnp.int32),
            pltpu.VMEM((BA,), jnp.int32),
            pltpu.VMEM((BA,), jnp.float32),
        ],
    )
    def att(s_hbm, d_hbm, src_hbm, dst_hbm, p_hbm,
            s_v, d_v, src_v, dst_v, p_v):
        cid = lax.axis_index("c")
        sid = lax.axis_index("s")
        wid = sid * 2 + cid
        row0 = wid * NB
        pltpu.sync_copy(s_hbm, s_v)
        pltpu.sync_copy(d_hbm, d_v)

        def batch(bi, _):
            pltpu.sync_copy(src_hbm.at[row0 + bi], src_v)
            pltpu.sync_copy(dst_hbm.at[row0 + bi], dst_v)
            for g in range(BA // 16):
                si = src_v[pl.ds(g * 16, 16)]
                di = dst_v[pl.ds(g * 16, 16)]
                sv = plsc.load_gather(s_v, [si])
                dv = plsc.load_gather(d_v, [di])
                e = sv + dv
                e = jnp.where(e > 0.0, e, 0.2 * e)
                p_v[pl.ds(g * 16, 16)] = jnp.exp(e)
            pltpu.sync_copy(p_v, p_hbm.at[row0 + bi])
            return ()
        lax.fori_loop(0, NB, batch, ())

    return att(s, d, src2, dst2)


def _edge_aggregate(h_int, p2, src16_2, dst2):
    @functools.partial(
        pl.kernel,
        out_type=[jax.ShapeDtypeStruct((NP * NCHUNK, CW), jnp.float32),
                  jax.ShapeDtypeStruct((NP * NCHUNK, CW), jnp.float32)],
        mesh=_sc_mesh(),
        compiler_params=_SC_PARAMS,
        scratch_types=[
            pltpu.VMEM((NB, BA), jnp.int32),
            pltpu.VMEM((NB, BA), jnp.int32),
            pltpu.VMEM((NB, BA), jnp.float32),
            pltpu.VMEM((BA, CW), jnp.float32),
            pltpu.VMEM((BA, CW), jnp.float32),
            pltpu.VMEM((BA, CW), jnp.float32),
            pltpu.VMEM((BA, CW), jnp.float32),
            pltpu.VMEM((BA,), jnp.int32),
            pltpu.VMEM_SHARED((NP, CW), jnp.float32),
            pltpu.SemaphoreType.DMA,
            pltpu.SemaphoreType.DMA,
        ],
    )
    def agg(h_hbm, src_hbm, dst_hbm, p_hbm, u0_hbm, u1_hbm,
            src_v, dst_v, p_v, rows0_v, rows1_v, zz_v, stage_v, idx_v,
            u_sp, sem0, sem1):
        cid = lax.axis_index("c")
        sid = lax.axis_index("s")
        wid = sid * 2 + cid
        row0 = wid * NB
        lane = lax.iota(jnp.int32, 16)
        hi8 = (lane >= 8).astype(jnp.int32)
        cidx = lane % 8
        pltpu.sync_copy(src_hbm.at[pl.ds(row0, NB)], src_v)
        pltpu.sync_copy(dst_hbm.at[pl.ds(row0, NB)], dst_v)
        pltpu.sync_copy(p_hbm.at[pl.ds(row0, NB)], p_v)
        zero16 = jnp.zeros((16,), jnp.float32)
        for k in range(BA // 2):
            plsc.store_scatter(zz_v, [2 * k + hi8, cidx], zero16)

        def scale(bi, rows_v):
            bvec = jnp.full((16,), bi, jnp.int32)
            for k in range(BA // 2):
                ridx = 2 * k + hi8
                pv = plsc.load_gather(p_v, [bvec, ridx])
                v = plsc.load_gather(rows_v, [ridx, cidx])
                plsc.store_scatter(rows_v, [ridx, cidx], v * pv)

        def chunk(c, _):
            def zinit(i, _):
                pltpu.sync_copy(zz_v, u_sp.at[pl.ds(sid * NPT + i * BA, BA)])
                return ()
            lax.fori_loop(0, NPT // BA, zinit, ())
            plsc.subcore_barrier()

            pltpu.async_copy(h_hbm.at[src_v.at[0]], rows0_v, sem0)

            def pair(k, _):
                b0 = 2 * k
                pltpu.async_copy(h_hbm.at[src_v.at[b0 + 1]], rows1_v, sem1)
                pltpu.make_async_copy(h_hbm.at[src_v.at[b0]], rows0_v,
                                      sem0).wait()
                scale(b0, rows0_v)
                pltpu.sync_copy(rows0_v, u_sp.at[dst_v.at[b0]], add=True)
                nxt = jnp.where(b0 + 2 >= NB, 0, b0 + 2)
                pltpu.async_copy(h_hbm.at[src_v.at[nxt]], rows0_v, sem0)
                pltpu.make_async_copy(h_hbm.at[src_v.at[b0 + 1]], rows1_v,
                                      sem1).wait()
                scale(b0 + 1, rows1_v)
                pltpu.sync_copy(rows1_v, u_sp.at[dst_v.at[b0 + 1]], add=True)
                return ()
            lax.fori_loop(0, NB // 2, pair, ())
            pltpu.make_async_copy(h_hbm.at[src_v.at[0]], rows0_v,
                                  sem0).wait()
            plsc.subcore_barrier()

            def dump(i, _):
                node0 = sid * NPT + i * BA
                pltpu.sync_copy(u_sp.at[pl.ds(node0, BA)], stage_v)
                for g in range(BA // 16):
                    idx_v[pl.ds(g * 16, 16)] = (
                        (node0 + g * 16 + lane) * NCHUNK + c)
                @pl.when(cid == 0)
                def _():
                    pltpu.sync_copy(stage_v, u0_hbm.at[idx_v])
                @pl.when(cid == 1)
                def _():
                    pltpu.sync_copy(stage_v, u1_hbm.at[idx_v])
                return ()
            lax.fori_loop(0, NPT // BA, dump, ())
            plsc.subcore_barrier()

            def bump(i, _):
                r = i // (BA // 16)
                g = i % (BA // 16)
                src_v[r, pl.ds(g * 16, 16)] = src_v[r, pl.ds(g * 16, 16)] + 1
                return ()
            lax.fori_loop(0, NB * (BA // 16), bump, ())
            return ()
        lax.fori_loop(0, NCHUNK, chunk, ())

    return agg(h_int, src16_2, dst2, p2)


def kernel(x, edge_index, W1, asrc1, adst1, b1, W2, asrc2, adst2, b2,
           Wf1, bf1, Wf2, bf2):
    f32 = jnp.float32
    x_pad = jnp.zeros((NP, FP), f32).at[:N, :F].set(x)
    W1p = jnp.zeros((FP, FP), f32).at[:F, :F].set(W1)
    W2p = jnp.zeros((FP, FP), f32).at[:F, :F].set(W2)
    A1 = jnp.zeros((FP, 8), f32).at[:F, 0].set(asrc1).at[:F, 1].set(adst1)
    A2 = jnp.zeros((FP, 8), f32).at[:F, 0].set(asrc2).at[:F, 1].set(adst2)
    b1p = jnp.zeros((1, FP), f32).at[0, :F].set(b1)
    b2p = jnp.zeros((1, FP), f32).at[0, :F].set(b2)
    Wf1p = jnp.zeros((FP, H1), f32).at[:F, :].set(Wf1)
    bf1p = bf1.reshape(1, H1)
    Wf2p = jnp.zeros((H1, LP), f32).at[:, :L].set(Wf2)
    bf2p = jnp.full((1, LP), -1e30, f32).at[0, :L].set(bf2)

    npad = EP - E
    pad_idx = (N + (jnp.arange(npad, dtype=jnp.int32) % (NP - N)))
    srcp = jnp.concatenate([edge_index[0], pad_idx])
    src2 = srcp.reshape(EP // BA, BA)
    src16_2 = (srcp * NCHUNK).reshape(EP // BA, BA)
    dst2 = jnp.concatenate([edge_index[1], pad_idx]).reshape(EP // BA, BA)

    def ureshape(us):
        return (us[0].reshape(NP, FP), us[1].reshape(NP, FP))

    h1_full, sd1 = _tc1(x_pad, W1p, A1)
    p1 = _edge_attention(sd1[:, 0], sd1[:, 1], src2, dst2)
    U1 = _edge_aggregate(h1_full.reshape(NP * NCHUNK, CW), p1, src16_2, dst2)

    h2_full, sd2 = _tc_epi(ureshape(U1), sd1, h1_full, b1p, W2p, A2)
    p2 = _edge_attention(sd2[:, 0], sd2[:, 1], src2, dst2)
    U2 = _edge_aggregate(h2_full.reshape(NP * NCHUNK, CW), p2, src16_2, dst2)

    out_full = _tc_fin(ureshape(U2), sd2, h2_full, b2p, Wf1p, bf1p, Wf2p,
                       bf2p)
    return out_full[:N, :L]
